# baseline (speedup 1.0000x reference)
import jax
import jax.numpy as jnp
from jax.experimental import pallas as pl
from jax.experimental.pallas import tpu as pltpu

T = 2048
D = 1024
F = 4096
E = 8
TOP_K = 2

BT = 128
BF = 512
NJ = F // BF
NP = T * TOP_K
NT = (NP + E * (BT - 1) + BT - 1) // BT
P_MAX = NT * BT


def _ffn_kernel(ts_ref, te_ref, x_ref, w1_ref, w3_ref, w2_ref, o_ref):
    j = pl.program_id(0)
    e = pl.program_id(1)
    w1 = w1_ref[0].astype(jnp.bfloat16)
    w3 = w3_ref[0].astype(jnp.bfloat16)
    w2 = w2_ref[0].astype(jnp.bfloat16)
    dn = (((1,), (1,)), ((), ()))

    def body(t, carry):
        row = pl.multiple_of(t * BT, BT)
        x = x_ref[pl.ds(row, BT), :]
        h1 = jax.lax.dot_general(x, w1, dn, preferred_element_type=jnp.float32)
        h3 = jax.lax.dot_general(x, w3, dn, preferred_element_type=jnp.float32)
        act = h1 * jax.nn.sigmoid(h1) * h3
        oe = jax.lax.dot_general(act.astype(jnp.bfloat16), w2, dn,
                                 preferred_element_type=jnp.float32)

        @pl.when(j == 0)
        def _init():
            o_ref[pl.ds(row, BT), :] = oe

        @pl.when(j != 0)
        def _acc():
            o_ref[pl.ds(row, BT), :] += oe

        return carry

    jax.lax.fori_loop(ts_ref[e], te_ref[e], body, 0)


def _grouped_ffn(x_sorted, w1, w3, w2, tile_start, tile_end):
    grid_spec = pltpu.PrefetchScalarGridSpec(
        num_scalar_prefetch=2,
        grid=(NJ, E),
        in_specs=[
            pl.BlockSpec((P_MAX, D), lambda j, e, ts, te: (0, 0)),
            pl.BlockSpec((1, BF, D), lambda j, e, ts, te: (e, j, 0)),
            pl.BlockSpec((1, BF, D), lambda j, e, ts, te: (e, j, 0)),
            pl.BlockSpec((1, D, BF), lambda j, e, ts, te: (e, 0, j)),
        ],
        out_specs=pl.BlockSpec((P_MAX, D), lambda j, e, ts, te: (0, 0)),
    )
    return pl.pallas_call(
        _ffn_kernel,
        grid_spec=grid_spec,
        out_shape=jax.ShapeDtypeStruct((P_MAX, D), jnp.float32),
    )(tile_start, tile_end, x_sorted, w1, w3, w2)


@jax.jit
def kernel(hidden_states, router_logits, w1, w2, w3):
    probs = jax.nn.softmax(router_logits.astype(jnp.float32), axis=-1)
    topw, topi = jax.lax.top_k(probs, TOP_K)
    topw = topw / jnp.sum(topw, axis=-1, keepdims=True)

    e_flat = topi.reshape(-1).astype(jnp.int32)
    onehot = jax.nn.one_hot(e_flat, E, dtype=jnp.int32)
    csum = jnp.cumsum(onehot, axis=0)
    counts = csum[-1]
    tiles_per_e = (counts + BT - 1) // BT
    tile_end = jnp.cumsum(tiles_per_e).astype(jnp.int32)
    tile_start = (tile_end - tiles_per_e).astype(jnp.int32)
    seg_start = tile_start * BT
    rank = jnp.sum(csum * onehot, axis=1) - 1
    slots = seg_start[e_flat] + rank
    sorted_ids = jnp.zeros(P_MAX, jnp.int32).at[slots].set(
        jnp.arange(NP, dtype=jnp.int32) // TOP_K)
    pos = slots.reshape(T, TOP_K)

    x_sorted = hidden_states.astype(jnp.bfloat16)[sorted_ids]
    y = _grouped_ffn(x_sorted, w1, w3, w2, tile_start, tile_end)
    out = (y[pos[:, 0]] * topw[:, 0:1] + y[pos[:, 1]] * topw[:, 1:2])
    return out.astype(hidden_states.dtype)


# speedup vs baseline: 1.0191x; 1.0191x over previous
import jax
import jax.numpy as jnp
from jax.experimental import pallas as pl
from jax.experimental.pallas import tpu as pltpu

T = 2048
D = 1024
F = 4096
E = 8
TOP_K = 2

BT = 128
BF = 512
NJ = F // BF
NP = T * TOP_K
NT = (NP + E * (BT - 1) + BT - 1) // BT
P_MAX = NT * BT


def _ffn_kernel(ts_ref, te_ref, x_ref, w1_ref, w3_ref, w2_ref, o_ref):
    j = pl.program_id(0)
    e = pl.program_id(1)
    w1 = w1_ref[0]
    w3 = w3_ref[0]
    w2 = w2_ref[0]
    dn = (((1,), (1,)), ((), ()))

    def body(t, carry):
        row = pl.multiple_of(t * BT, BT)
        x = x_ref[pl.ds(row, BT), :]
        h1 = jax.lax.dot_general(x, w1, dn, preferred_element_type=jnp.float32)
        h3 = jax.lax.dot_general(x, w3, dn, preferred_element_type=jnp.float32)
        act = h1 * jax.nn.sigmoid(h1) * h3
        oe = jax.lax.dot_general(act, w2, dn,
                                 preferred_element_type=jnp.float32)

        @pl.when(j == 0)
        def _init():
            o_ref[pl.ds(row, BT), :] = oe

        @pl.when(j != 0)
        def _acc():
            o_ref[pl.ds(row, BT), :] += oe

        return carry

    jax.lax.fori_loop(ts_ref[e], te_ref[e], body, 0)


def _grouped_ffn(x_sorted, w1, w3, w2, tile_start, tile_end):
    grid_spec = pltpu.PrefetchScalarGridSpec(
        num_scalar_prefetch=2,
        grid=(NJ, E),
        in_specs=[
            pl.BlockSpec((P_MAX, D), lambda j, e, ts, te: (0, 0)),
            pl.BlockSpec((1, BF, D), lambda j, e, ts, te: (e, j, 0)),
            pl.BlockSpec((1, BF, D), lambda j, e, ts, te: (e, j, 0)),
            pl.BlockSpec((1, D, BF), lambda j, e, ts, te: (e, 0, j)),
        ],
        out_specs=pl.BlockSpec((P_MAX, D), lambda j, e, ts, te: (0, 0)),
    )
    return pl.pallas_call(
        _ffn_kernel,
        grid_spec=grid_spec,
        out_shape=jax.ShapeDtypeStruct((P_MAX, D), jnp.float32),
    )(tile_start, tile_end, x_sorted, w1, w3, w2)


@jax.jit
def kernel(hidden_states, router_logits, w1, w2, w3):
    probs = jax.nn.softmax(router_logits.astype(jnp.float32), axis=-1)
    topw, topi = jax.lax.top_k(probs, TOP_K)
    topw = topw / jnp.sum(topw, axis=-1, keepdims=True)

    e_flat = topi.reshape(-1).astype(jnp.int32)
    onehot = jax.nn.one_hot(e_flat, E, dtype=jnp.int32)
    csum = jnp.cumsum(onehot, axis=0)
    counts = csum[-1]
    tiles_per_e = (counts + BT - 1) // BT
    tile_end = jnp.cumsum(tiles_per_e).astype(jnp.int32)
    tile_start = (tile_end - tiles_per_e).astype(jnp.int32)
    seg_start = tile_start * BT
    rank = jnp.sum(csum * onehot, axis=1) - 1
    slots = seg_start[e_flat] + rank
    sorted_ids = jnp.zeros(P_MAX, jnp.int32).at[slots].set(
        jnp.arange(NP, dtype=jnp.int32) // TOP_K)
    pos = slots.reshape(T, TOP_K)

    x_sorted = hidden_states[sorted_ids]
    y = _grouped_ffn(x_sorted, w1, w3, w2, tile_start, tile_end)
    out = (y[pos[:, 0]] * topw[:, 0:1] + y[pos[:, 1]] * topw[:, 1:2])
    return out.astype(hidden_states.dtype)


# speedup vs baseline: 1.4845x; 1.4567x over previous
import jax
import jax.numpy as jnp
from jax.experimental import pallas as pl
from jax.experimental.pallas import tpu as pltpu

T = 2048
D = 1024
F = 4096
E = 8
TOP_K = 2

BT = 256
BF = 512
NJ = F // BF
NP = T * TOP_K
NT = (NP + E * (BT - 1) + BT - 1) // BT
P_MAX = NT * BT


def _ffn_kernel(ts_ref, te_ref, x_ref, w1_ref, w3_ref, w2_ref, o_ref):
    j = pl.program_id(0)
    e = pl.program_id(1)
    w1 = w1_ref[0].astype(jnp.bfloat16)
    w3 = w3_ref[0].astype(jnp.bfloat16)
    w2 = w2_ref[0].astype(jnp.bfloat16)
    dn = (((1,), (1,)), ((), ()))

    def body(t, carry):
        row = pl.multiple_of(t * BT, BT)
        x = x_ref[pl.ds(row, BT), :]
        h1 = jax.lax.dot_general(x, w1, dn, preferred_element_type=jnp.float32)
        h3 = jax.lax.dot_general(x, w3, dn, preferred_element_type=jnp.float32)
        act = h1 * jax.nn.sigmoid(h1) * h3
        oe = jax.lax.dot_general(act.astype(jnp.bfloat16), w2, dn,
                                 preferred_element_type=jnp.float32)

        @pl.when(j == 0)
        def _init():
            o_ref[pl.ds(row, BT), :] = oe

        @pl.when(j != 0)
        def _acc():
            o_ref[pl.ds(row, BT), :] += oe

        return carry

    jax.lax.fori_loop(ts_ref[e], te_ref[e], body, 0)


def _grouped_ffn(x_sorted, w1, w3, w2, tile_start, tile_end):
    grid_spec = pltpu.PrefetchScalarGridSpec(
        num_scalar_prefetch=2,
        grid=(NJ, E),
        in_specs=[
            pl.BlockSpec((P_MAX, D), lambda j, e, ts, te: (0, 0)),
            pl.BlockSpec((1, BF, D), lambda j, e, ts, te: (e, j, 0)),
            pl.BlockSpec((1, BF, D), lambda j, e, ts, te: (e, j, 0)),
            pl.BlockSpec((1, D, BF), lambda j, e, ts, te: (e, 0, j)),
        ],
        out_specs=pl.BlockSpec((P_MAX, D), lambda j, e, ts, te: (0, 0)),
    )
    return pl.pallas_call(
        _ffn_kernel,
        grid_spec=grid_spec,
        out_shape=jax.ShapeDtypeStruct((P_MAX, D), jnp.float32),
    )(tile_start, tile_end, x_sorted, w1, w3, w2)


@jax.jit
def kernel(hidden_states, router_logits, w1, w2, w3):
    probs = jax.nn.softmax(router_logits.astype(jnp.float32), axis=-1)
    topw, topi = jax.lax.top_k(probs, TOP_K)
    topw = topw / jnp.sum(topw, axis=-1, keepdims=True)

    e_flat = topi.reshape(-1).astype(jnp.int32)
    onehot = jax.nn.one_hot(e_flat, E, dtype=jnp.int32)
    csum = jnp.cumsum(onehot, axis=0)
    counts = csum[-1]
    tiles_per_e = (counts + BT - 1) // BT
    tile_end = jnp.cumsum(tiles_per_e).astype(jnp.int32)
    tile_start = (tile_end - tiles_per_e).astype(jnp.int32)
    seg_start = tile_start * BT
    rank = jnp.sum(csum * onehot, axis=1) - 1
    slots = seg_start[e_flat] + rank
    sorted_ids = jnp.zeros(P_MAX, jnp.int32).at[slots].set(
        jnp.arange(NP, dtype=jnp.int32) // TOP_K)
    pos = slots.reshape(T, TOP_K)

    x_sorted = hidden_states.astype(jnp.bfloat16)[sorted_ids]
    y = _grouped_ffn(x_sorted, w1, w3, w2, tile_start, tile_end)
    out = (y[pos[:, 0]] * topw[:, 0:1] + y[pos[:, 1]] * topw[:, 1:2])
    return out.astype(hidden_states.dtype)
